# baseline (speedup 1.0000x reference)
import jax
import jax.numpy as jnp
from jax.experimental import pallas as pl

_NUM_EXPERTS = 64
_TOP_K = 8
_BT = 512
_SUB = 512


def _router_kernel(x_ref, wt_ref, logits_ref, vals_ref, idx_ref):
    x = x_ref[...]
    wt = wt_ref[...]
    logits = jnp.dot(x, wt, preferred_element_type=jnp.float32)
    logits_ref[...] = logits

    for c in range(_BT // _SUB):
        cols = pl.ds(c * _SUB, _SUB)
        work = logits_ref[cols, :].T
        row = jax.lax.broadcasted_iota(jnp.int32, work.shape, 0)
        vals = []
        idxs = []
        for _ in range(_TOP_K):
            m = jnp.max(work, axis=0, keepdims=True)
            i = jnp.argmax(work, axis=0).astype(jnp.int32)[None, :]
            vals.append(m)
            idxs.append(i)
            work = jnp.where(row == i, -jnp.inf, work)
        top_vals = jnp.concatenate(vals, axis=0)
        top_idx = jnp.concatenate(idxs, axis=0)
        e = jnp.exp(top_vals - top_vals[0:1, :])
        vals_ref[cols, :] = (e / jnp.sum(e, axis=0, keepdims=True)).T
        idx_ref[cols, :] = top_idx.T


@jax.jit
def kernel(hidden_states, weight):
    tokens, hidden = hidden_states.shape
    wt = weight.T
    grid = (tokens // _BT,)
    out = pl.pallas_call(
        _router_kernel,
        grid=grid,
        in_specs=[
            pl.BlockSpec((_BT, hidden), lambda i: (i, 0)),
            pl.BlockSpec((hidden, _NUM_EXPERTS), lambda i: (0, 0)),
        ],
        out_specs=[
            pl.BlockSpec((_BT, _NUM_EXPERTS), lambda i: (i, 0)),
            pl.BlockSpec((_BT, _TOP_K), lambda i: (i, 0)),
            pl.BlockSpec((_BT, _TOP_K), lambda i: (i, 0)),
        ],
        out_shape=[
            jax.ShapeDtypeStruct((tokens, _NUM_EXPERTS), jnp.float32),
            jax.ShapeDtypeStruct((tokens, _TOP_K), jnp.float32),
            jax.ShapeDtypeStruct((tokens, _TOP_K), jnp.int32),
        ],
    )(hidden_states, wt)
    return tuple(out)


# speedup vs baseline: 1.1874x; 1.1874x over previous
import jax
import jax.numpy as jnp
from jax.experimental import pallas as pl
from jax.experimental.pallas import tpu as pltpu

_NUM_EXPERTS = 64
_TOP_K = 8
_BT = 2048
_SUB = 512
_S = 4
_CH = _BT // _S


def _x_copy(x_hbm, xbuf, insem, t, s):
    slot = jax.lax.rem(t, 2)
    return pltpu.make_async_copy(
        x_hbm.at[pl.ds(t * _BT + s * _CH, _CH), :],
        xbuf.at[slot, pl.ds(s * _CH, _CH), :],
        insem.at[slot, s],
    )


def _router_kernel(x_hbm, wt_ref, logits_ref, vals_ref, idx_ref, xbuf, insem):
    i = pl.program_id(0)
    nt = pl.num_programs(0)

    @pl.when(i == 0)
    def _():
        for s in range(_S):
            _x_copy(x_hbm, xbuf, insem, i, s).start()

    @pl.when(i + 1 < nt)
    def _():
        for s in range(_S):
            _x_copy(x_hbm, xbuf, insem, i + 1, s).start()

    for s in range(_S):
        _x_copy(x_hbm, xbuf, insem, i, s).wait()

    slot = jax.lax.rem(i, 2)
    x = xbuf[slot]
    logits = jnp.dot(x, wt_ref[...], preferred_element_type=jnp.float32)
    logits_ref[...] = logits

    for c in range(_BT // _SUB):
        cols = pl.ds(c * _SUB, _SUB)
        work = logits_ref[cols, :].T
        row = jax.lax.broadcasted_iota(jnp.int32, work.shape, 0)
        vals = []
        idxs = []
        for _ in range(_TOP_K):
            m = jnp.max(work, axis=0, keepdims=True)
            ai = jnp.argmax(work, axis=0).astype(jnp.int32)[None, :]
            vals.append(m)
            idxs.append(ai)
            work = jnp.where(row == ai, -jnp.inf, work)
        top_vals = jnp.concatenate(vals, axis=0)
        top_idx = jnp.concatenate(idxs, axis=0)
        e = jnp.exp(top_vals - top_vals[0:1, :])
        vals_ref[cols, :] = (e / jnp.sum(e, axis=0, keepdims=True)).T
        idx_ref[cols, :] = top_idx.T


@jax.jit
def kernel(hidden_states, weight):
    tokens, hidden = hidden_states.shape
    wt = weight.T
    grid = (tokens // _BT,)
    out = pl.pallas_call(
        _router_kernel,
        grid=grid,
        in_specs=[
            pl.BlockSpec(memory_space=pl.ANY),
            pl.BlockSpec((hidden, _NUM_EXPERTS), lambda i: (0, 0)),
        ],
        out_specs=[
            pl.BlockSpec((_BT, _NUM_EXPERTS), lambda i: (i, 0)),
            pl.BlockSpec((_BT, _TOP_K), lambda i: (i, 0)),
            pl.BlockSpec((_BT, _TOP_K), lambda i: (i, 0)),
        ],
        out_shape=[
            jax.ShapeDtypeStruct((tokens, _NUM_EXPERTS), jnp.float32),
            jax.ShapeDtypeStruct((tokens, _TOP_K), jnp.float32),
            jax.ShapeDtypeStruct((tokens, _TOP_K), jnp.int32),
        ],
        scratch_shapes=[
            pltpu.VMEM((2, _BT, hidden), jnp.float32),
            pltpu.SemaphoreType.DMA((2, _S)),
        ],
    )(hidden_states, wt)
    return tuple(out)


# speedup vs baseline: 1.2120x; 1.0207x over previous
import jax
import jax.numpy as jnp
from jax.experimental import pallas as pl
from jax.experimental.pallas import tpu as pltpu

_NUM_EXPERTS = 64
_TOP_K = 8
_BT = 2048
_SUB = 512
_S = 4
_CH = _BT // _S


def _x_copy(x_hbm, xbuf, insem, t, s):
    slot = jax.lax.rem(t, 2)
    return pltpu.make_async_copy(
        x_hbm.at[pl.ds(t * _BT + s * _CH, _CH), :],
        xbuf.at[slot, pl.ds(s * _CH, _CH), :],
        insem.at[slot, s],
    )


def _router_kernel(x_hbm, w_ref, logits_ref, vals_ref, idx_ref, xbuf, insem):
    i = pl.program_id(0)
    nt = pl.num_programs(0)

    @pl.when(i == 0)
    def _():
        for s in range(_S):
            _x_copy(x_hbm, xbuf, insem, i, s).start()

    @pl.when(i + 1 < nt)
    def _():
        for s in range(_S):
            _x_copy(x_hbm, xbuf, insem, i + 1, s).start()

    for s in range(_S):
        _x_copy(x_hbm, xbuf, insem, i, s).wait()

    slot = jax.lax.rem(i, 2)
    x = xbuf[slot]
    logits = jax.lax.dot_general(
        x, w_ref[...], (((1,), (1,)), ((), ())),
        preferred_element_type=jnp.float32)
    logits_ref[...] = logits

    for c in range(_BT // _SUB):
        cols = pl.ds(c * _SUB, _SUB)
        work = logits_ref[cols, :].T
        row = jax.lax.broadcasted_iota(jnp.int32, work.shape, 0)
        vals = []
        idxs = []
        for _ in range(_TOP_K):
            m = jnp.max(work, axis=0, keepdims=True)
            ai = jnp.argmax(work, axis=0).astype(jnp.int32)[None, :]
            vals.append(m)
            idxs.append(ai)
            work = jnp.where(row == ai, -jnp.inf, work)
        top_vals = jnp.concatenate(vals, axis=0)
        top_idx = jnp.concatenate(idxs, axis=0)
        e = jnp.exp(top_vals - top_vals[0:1, :])
        vals_ref[cols, :] = (e / jnp.sum(e, axis=0, keepdims=True)).T
        idx_ref[cols, :] = top_idx.T


@jax.jit
def kernel(hidden_states, weight):
    tokens, hidden = hidden_states.shape
    grid = (tokens // _BT,)
    out = pl.pallas_call(
        _router_kernel,
        grid=grid,
        in_specs=[
            pl.BlockSpec(memory_space=pl.ANY),
            pl.BlockSpec((_NUM_EXPERTS, hidden), lambda i: (0, 0)),
        ],
        out_specs=[
            pl.BlockSpec((_BT, _NUM_EXPERTS), lambda i: (i, 0)),
            pl.BlockSpec((_BT, _TOP_K), lambda i: (i, 0)),
            pl.BlockSpec((_BT, _TOP_K), lambda i: (i, 0)),
        ],
        out_shape=[
            jax.ShapeDtypeStruct((tokens, _NUM_EXPERTS), jnp.float32),
            jax.ShapeDtypeStruct((tokens, _TOP_K), jnp.float32),
            jax.ShapeDtypeStruct((tokens, _TOP_K), jnp.int32),
        ],
        scratch_shapes=[
            pltpu.VMEM((2, _BT, hidden), jnp.float32),
            pltpu.SemaphoreType.DMA((2, _S)),
        ],
    )(hidden_states, weight)
    return tuple(out)


# speedup vs baseline: 1.2226x; 1.0088x over previous
import jax
import jax.numpy as jnp
from jax.experimental import pallas as pl

_NUM_EXPERTS = 64
_TOP_K = 8
_BT = 2048
_SUB = 512


def _router_kernel(x_ref, w_ref, logits_ref, vals_ref, idx_ref):
    logits = jax.lax.dot_general(
        x_ref[...], w_ref[...], (((1,), (1,)), ((), ())),
        preferred_element_type=jnp.float32)
    logits_ref[...] = logits

    for c in range(_BT // _SUB):
        cols = pl.ds(c * _SUB, _SUB)
        work = logits_ref[cols, :].T
        row = jax.lax.broadcasted_iota(jnp.int32, work.shape, 0)
        vals = []
        idxs = []
        for _ in range(_TOP_K):
            m = jnp.max(work, axis=0, keepdims=True)
            ai = jnp.argmax(work, axis=0).astype(jnp.int32)[None, :]
            vals.append(m)
            idxs.append(ai)
            work = jnp.where(row == ai, -jnp.inf, work)
        top_vals = jnp.concatenate(vals, axis=0)
        top_idx = jnp.concatenate(idxs, axis=0)
        e = jnp.exp(top_vals - top_vals[0:1, :])
        vals_ref[cols, :] = (e / jnp.sum(e, axis=0, keepdims=True)).T
        idx_ref[cols, :] = top_idx.T


@jax.jit
def kernel(hidden_states, weight):
    tokens, hidden = hidden_states.shape
    grid = (tokens // _BT,)
    out = pl.pallas_call(
        _router_kernel,
        grid=grid,
        in_specs=[
            pl.BlockSpec((_BT, hidden), lambda i: (i, 0)),
            pl.BlockSpec((_NUM_EXPERTS, hidden), lambda i: (0, 0)),
        ],
        out_specs=[
            pl.BlockSpec((_BT, _NUM_EXPERTS), lambda i: (i, 0)),
            pl.BlockSpec((_BT, _TOP_K), lambda i: (i, 0)),
            pl.BlockSpec((_BT, _TOP_K), lambda i: (i, 0)),
        ],
        out_shape=[
            jax.ShapeDtypeStruct((tokens, _NUM_EXPERTS), jnp.float32),
            jax.ShapeDtypeStruct((tokens, _TOP_K), jnp.float32),
            jax.ShapeDtypeStruct((tokens, _TOP_K), jnp.int32),
        ],
    )(hidden_states, weight)
    return tuple(out)
